# baseline (speedup 1.0000x reference)
import jax
import jax.numpy as jnp
from jax import lax
from jax.experimental import pallas as pl
from jax.experimental.pallas import tpu as pltpu
from jax.experimental.pallas import tpu_sc as plsc

HIDDEN = 768
BATCH = 32
SEQ = 512
EPS = 1e-12

NC = 2
NS = 16
NW = NC * NS
SBLK = SEQ // NW
LANES = 16
NCHUNK = HIDDEN // LANES


def _ln_body(ids_hbm, word_hbm, pos_hbm, gamma_hbm, beta_hbm, out_hbm,
             idx_v, rows_v, pos_v, g_v, b_v, gsem):
    wid = lax.axis_index("s") * NC + lax.axis_index("c")
    s0 = wid * SBLK

    pltpu.sync_copy(pos_hbm.at[pl.ds(s0, SBLK)], pos_v)
    pltpu.sync_copy(gamma_hbm, g_v)
    pltpu.sync_copy(beta_hbm, b_v)

    def load_ids(b, c):
        pltpu.sync_copy(ids_hbm.at[b, pl.ds(s0, SBLK)], idx_v.at[b])
        return c
    lax.fori_loop(0, BATCH, load_ids, 0)

    zero = jnp.zeros((LANES,), jnp.float32)
    lane = lax.iota(jnp.int32, LANES)
    rot_idx = [(lane + sh) & (LANES - 1) for sh in (8, 4, 2, 1)]

    def allsum(x):
        for idx in rot_idx:
            x = x + x.at[idx].get(mode="promise_in_bounds")
        return x

    def per_batch(b, c):
        pltpu.async_copy(word_hbm.at[idx_v.at[b]], rows_v, gsem).wait()

        def p1(k, carry):
            sums, sqs = carry
            col = k * LANES
            ns, nq = [], []
            for r in range(SBLK):
                x = rows_v[r, pl.ds(col, LANES)] + pos_v[r, pl.ds(col, LANES)]
                rows_v[r, pl.ds(col, LANES)] = x
                ns.append(sums[r] + x)
                nq.append(sqs[r] + x * x)
            return tuple(ns), tuple(nq)

        sums, sqs = lax.fori_loop(
            0, NCHUNK, p1,
            (tuple([zero] * SBLK), tuple([zero] * SBLK)))

        aa, cc = [], []
        for r in range(SBLK):
            mean = allsum(sums[r]) * (1.0 / HIDDEN)
            var = allsum(sqs[r]) * (1.0 / HIDDEN) - mean * mean + EPS
            i = lax.bitcast_convert_type(var, jnp.int32)
            i = 0x5F3759DF - lax.shift_right_arithmetic(i, 1)
            y = lax.bitcast_convert_type(i, jnp.float32)
            for _ in range(3):
                y = y * (1.5 - 0.5 * var * y * y)
            aa.append(y)
            cc.append(mean * y)

        def p2(k, c2):
            col = k * LANES
            g = g_v[pl.ds(col, LANES)]
            bb = b_v[pl.ds(col, LANES)]
            for r in range(SBLK):
                x = rows_v[r, pl.ds(col, LANES)]
                rows_v[r, pl.ds(col, LANES)] = (x * aa[r] - cc[r]) * g + bb
            return c2
        lax.fori_loop(0, NCHUNK, p2, 0)

        pltpu.sync_copy(rows_v, out_hbm.at[b, pl.ds(s0, SBLK)])
        return c
    lax.fori_loop(0, BATCH, per_batch, 0)


def kernel(input_ids, word_table, pos_table, ln_gamma, ln_beta):
    ids = input_ids.astype(jnp.int32)
    f = pl.kernel(
        _ln_body,
        out_type=jax.ShapeDtypeStruct((BATCH, SEQ, HIDDEN), jnp.float32),
        mesh=plsc.VectorSubcoreMesh(core_axis_name="c", subcore_axis_name="s"),
        scratch_types=[
            pltpu.VMEM((BATCH, SBLK), jnp.int32),
            pltpu.VMEM((SBLK, HIDDEN), jnp.float32),
            pltpu.VMEM((SBLK, HIDDEN), jnp.float32),
            pltpu.VMEM((HIDDEN,), jnp.float32),
            pltpu.VMEM((HIDDEN,), jnp.float32),
            pltpu.SemaphoreType.DMA,
        ],
    )
    return f(ids, word_table, pos_table, ln_gamma, ln_beta)
